# baseline (speedup 1.0000x reference)
import functools

import jax
import jax.numpy as jnp
from jax import lax
from jax.experimental import pallas as pl
from jax.experimental.pallas import tpu as pltpu
from jax.experimental.pallas import tpu_sc as plsc

N = 10000
E = 320000
F = 128
H = 128

NC = 2
NS = 16
NP = 10240
CH = 48
NCH = 216
PH = 2
NCHP = NCH // PH
NBUF = 4
EP = NC * NS * NCH * CH
EPC = EP // NC
EPT = EPC // NS
EPP = EPT // PH
RP = NP // NS
LANES = 16
FG = F // LANES


def _sc_agg(x, srcb, dstb, w, z):
    mesh = plsc.VectorSubcoreMesh(core_axis_name="c", subcore_axis_name="s")

    @functools.partial(
        pl.kernel,
        out_type=jax.ShapeDtypeStruct((NC, NP, F), jnp.float32),
        mesh=mesh,
        compiler_params=pltpu.CompilerParams(use_tc_tiling_on_sc=False),
        scratch_types=[
            pltpu.VMEM_SHARED((NP, F), jnp.float32),
            pltpu.VMEM((NCHP, CH), jnp.int32),
            pltpu.VMEM((NCHP, CH), jnp.int32),
            pltpu.VMEM((EPP,), jnp.float32),
            [pltpu.VMEM((CH, F), jnp.float32)] * NBUF,
            [pltpu.SemaphoreType.DMA] * NBUF,
            [pltpu.SemaphoreType.DMA] * NBUF,
        ],
    )
    def k(x_hbm, src_hbm, dst_hbm, w_hbm, z_hbm, out_hbm, acc, src_v, dst_v,
          w_v, rows, gsem, ssem):
        c = lax.axis_index("c")
        s = lax.axis_index("s")
        tid = (1 - c) * NS + s
        arow = s * RP

        pltpu.sync_copy(z_hbm, acc.at[pl.ds(arow, RP)])
        plsc.subcore_barrier()

        def scale(r, i):
            def grp(g, carry):
                w16 = w_v[pl.ds(i * CH + g * LANES, LANES)]
                for em in range(LANES):
                    we = jnp.broadcast_to(w16[em:em + 1], (LANES,))
                    e = g * LANES + em
                    for f in range(FG):
                        sl = pl.ds(f * LANES, LANES)
                        r[e, sl] = r[e, sl] * we
                return carry
            lax.fori_loop(0, CH // LANES, grp, 0)

        def gather(i, k):
            pltpu.async_copy(x_hbm.at[src_v.at[i]], rows[k], gsem[k])

        def gather_wait(i, k):
            pltpu.make_async_copy(x_hbm.at[src_v.at[i]], rows[k],
                                  gsem[k]).wait()

        def scatter(i, k):
            pltpu.async_copy(rows[k], acc.at[dst_v.at[i]], ssem[k], add=True)

        def scatter_wait(i, k):
            pltpu.make_async_copy(rows[k], acc.at[dst_v.at[i]],
                                  ssem[k]).wait()

        def phase(ph, carry):
            pbase = tid * NCH + ph * NCHP
            pltpu.sync_copy(src_hbm.at[pl.ds(pbase, NCHP)], src_v)
            pltpu.sync_copy(dst_hbm.at[pl.ds(pbase, NCHP)], dst_v)
            pltpu.sync_copy(w_hbm.at[pl.ds(tid * EPT + ph * EPP, EPP)], w_v)
            for k in range(NBUF):
                gather(k, k)

            def body(j, carry2):
                for k in range(NBUF):
                    i = NBUF * j + k
                    gather_wait(i, k)
                    scale(rows[k], i)
                    scatter(i, k)
                for k in range(NBUF):
                    i = NBUF * j + k
                    scatter_wait(i, k)

                    @pl.when(i + NBUF < NCHP)
                    def _():
                        gather(i + NBUF, k)

                return carry2

            lax.fori_loop(0, NCHP // NBUF, body, 0)
            return carry

        lax.fori_loop(0, PH, phase, 0)
        plsc.subcore_barrier()

        pltpu.sync_copy(acc.at[pl.ds(arow, RP)], out_hbm.at[c, pl.ds(arow, RP)])

    return k(x, srcb, dstb, w, z)


def _combine(p, W, b):
    BR = 1000

    def body(p0_ref, p1_ref, w_ref, b_ref, o_ref):
        acc = p0_ref[0] + p1_ref[0]
        o_ref[...] = (
            jnp.dot(acc, w_ref[...], preferred_element_type=jnp.float32)
            + b_ref[...]
        )

    return pl.pallas_call(
        body,
        grid=(N // BR,),
        in_specs=[
            pl.BlockSpec((1, BR, F), lambda i: (0, i, 0)),
            pl.BlockSpec((1, BR, F), lambda i: (1, i, 0)),
            pl.BlockSpec((F, H), lambda i: (0, 0)),
            pl.BlockSpec((1, H), lambda i: (0, 0)),
        ],
        out_specs=pl.BlockSpec((BR, H), lambda i: (i, 0)),
        out_shape=jax.ShapeDtypeStruct((N, H), jnp.float32),
    )(p, p, W, b.reshape(1, H))


def kernel(x, edge_index, edge_weight, W, b):
    src = edge_index[0].astype(jnp.int32)
    dst = edge_index[1].astype(jnp.int32)
    pad = EP - E
    srcb = jnp.concatenate([src, jnp.zeros((pad,), jnp.int32)]).reshape(
        EP // CH, CH)
    pad_dst = jnp.arange(pad, dtype=jnp.int32) % N
    dstb = jnp.concatenate([dst, pad_dst]).reshape(EP // CH, CH)
    wp = jnp.concatenate([edge_weight, jnp.zeros((pad,), jnp.float32)])
    z = jnp.zeros((RP, F), jnp.float32)
    p = _sc_agg(x, srcb, dstb, wp, z)
    return _combine(p, W, b)


# speedup vs baseline: 3.2577x; 3.2577x over previous
import functools

import jax
import jax.numpy as jnp
from jax import lax
from jax.experimental import pallas as pl
from jax.experimental.pallas import tpu as pltpu
from jax.experimental.pallas import tpu_sc as plsc

N = 10000
E = 320000
F = 128
H = 128

NC = 2
NS = 16
NP = 10240
CH = 48
NCH = 216
PH = 2
NCHP = NCH // PH
NBUF = 4
EP = NC * NS * NCH * CH
EPC = EP // NC
EPT = EPC // NS
EPP = EPT // PH
RP = NP // NS
LANES = 16
FG = F // LANES


def _sc_agg(x, srcb, dstb, w, z):
    mesh = plsc.VectorSubcoreMesh(core_axis_name="c", subcore_axis_name="s")

    @functools.partial(
        pl.kernel,
        out_type=jax.ShapeDtypeStruct((NC, NP, F), jnp.float32),
        mesh=mesh,
        compiler_params=pltpu.CompilerParams(use_tc_tiling_on_sc=False),
        scratch_types=[
            pltpu.VMEM_SHARED((NP, F), jnp.float32),
            pltpu.VMEM((NCHP, CH), jnp.int32),
            pltpu.VMEM((NCHP, CH), jnp.int32),
            pltpu.VMEM((EPP,), jnp.float32),
            [pltpu.VMEM((CH, F), jnp.float32)] * NBUF,
            [pltpu.SemaphoreType.DMA] * NBUF,
            [pltpu.SemaphoreType.DMA] * NBUF,
        ],
    )
    def k(x_hbm, src_hbm, dst_hbm, w_hbm, z_hbm, out_hbm, acc, src_v, dst_v,
          w_v, rows, gsem, ssem):
        c = lax.axis_index("c")
        s = lax.axis_index("s")
        tid = c * NS + s
        arow = s * RP

        pltpu.sync_copy(z_hbm, acc.at[pl.ds(arow, RP)])
        plsc.subcore_barrier()

        def scale(r, i):
            def grp(g, carry):
                w16 = w_v[pl.ds(i * CH + g * LANES, LANES)]
                for em in range(LANES):
                    we = jnp.broadcast_to(w16[em:em + 1], (LANES,))
                    e = g * LANES + em
                    for f in range(FG):
                        sl = pl.ds(f * LANES, LANES)
                        r[e, sl] = r[e, sl] * we
                return carry
            lax.fori_loop(0, CH // LANES, grp, 0)

        def gather(i, k):
            pltpu.async_copy(x_hbm.at[src_v.at[i]], rows[k], gsem[k])

        def gather_wait(i, k):
            pltpu.make_async_copy(x_hbm.at[src_v.at[i]], rows[k],
                                  gsem[k]).wait()

        def scatter(i, k):
            pltpu.async_copy(rows[k], acc.at[dst_v.at[i]], ssem[k], add=True)

        def scatter_wait(i, k):
            pltpu.make_async_copy(rows[k], acc.at[dst_v.at[i]],
                                  ssem[k]).wait()

        def phase(ph, carry):
            pbase = tid * NCH + ph * NCHP
            pltpu.sync_copy(src_hbm.at[pl.ds(pbase, NCHP)], src_v)
            pltpu.sync_copy(dst_hbm.at[pl.ds(pbase, NCHP)], dst_v)
            pltpu.sync_copy(w_hbm.at[pl.ds(tid * EPT + ph * EPP, EPP)], w_v)
            for k in range(NBUF):
                gather(k, k)

            def body(j, carry2):
                for k in range(NBUF):
                    i = NBUF * j + k
                    gather_wait(i, k)
                    scale(rows[k], i)
                    scatter(i, k)
                for k in range(NBUF):
                    i = NBUF * j + k
                    scatter_wait(i, k)

                    @pl.when(i + NBUF < NCHP)
                    def _():
                        gather(i + NBUF, k)

                return carry2

            lax.fori_loop(0, NCHP // NBUF, body, 0)
            return carry

        lax.fori_loop(0, PH, phase, 0)
        plsc.subcore_barrier()

        pltpu.sync_copy(acc.at[pl.ds(arow, RP)], out_hbm.at[c, pl.ds(arow, RP)])

    return k(x, srcb, dstb, w, z)


def _combine(p, W, b):
    BR = 1000

    def body(p0_ref, p1_ref, w_ref, b_ref, o_ref):
        acc = p0_ref[0] + p1_ref[0]
        o_ref[...] = (
            jnp.dot(acc, w_ref[...], preferred_element_type=jnp.float32)
            + b_ref[...]
        )

    return pl.pallas_call(
        body,
        grid=(N // BR,),
        in_specs=[
            pl.BlockSpec((1, BR, F), lambda i: (0, i, 0)),
            pl.BlockSpec((1, BR, F), lambda i: (1, i, 0)),
            pl.BlockSpec((F, H), lambda i: (0, 0)),
            pl.BlockSpec((1, H), lambda i: (0, 0)),
        ],
        out_specs=pl.BlockSpec((BR, H), lambda i: (i, 0)),
        out_shape=jax.ShapeDtypeStruct((N, H), jnp.float32),
    )(p, p, W, b.reshape(1, H))


def kernel(x, edge_index, edge_weight, W, b):
    src = edge_index[0].astype(jnp.int32)
    dst = edge_index[1].astype(jnp.int32)
    pad = EP - E
    pad_idx = jnp.arange(pad, dtype=jnp.int32) % N
    srcb = jnp.concatenate([src, pad_idx]).reshape(EP // CH, CH)
    dstb = jnp.concatenate([dst, pad_idx]).reshape(EP // CH, CH)
    wp = jnp.concatenate([edge_weight, jnp.zeros((pad,), jnp.float32)])
    z = jnp.zeros((RP, F), jnp.float32)
    p = _sc_agg(x, srcb, dstb, wp, z)
    return _combine(p, W, b)
